# baseline (speedup 1.0000x reference)
import functools

import jax
import jax.numpy as jnp
from jax import lax
from jax.experimental import pallas as pl
from jax.experimental.pallas import tpu as pltpu
from jax.experimental.pallas import tpu_sc as plsc

_NW = 32
_CC = 14
_R = 112
_BN = 1000


def _argmin_body(p_ref, muT_ref, idx_ref):
    p = p_ref[...]
    muT = muT_ref[...]
    k = muT.shape[1]
    m2 = jnp.sum(muT * muT, axis=0, keepdims=True)
    p2 = jnp.sum(p * p, axis=1, keepdims=True)
    dotn = jnp.dot(p, -2.0 * muT, preferred_element_type=jnp.float32)
    d2 = jnp.maximum((p2 + m2) + dotn, 0.0)
    mind = jnp.min(d2, axis=1, keepdims=True)
    ids = lax.broadcasted_iota(jnp.int32, d2.shape, 1)
    idx = jnp.min(jnp.where(d2 == mind, ids, k), axis=1, keepdims=True)
    idx_ref[...] = jnp.minimum(idx, k - 1).astype(jnp.int32)


def _ctable_body(h_ref, rho_ref, w1_ref, b1_ref, c_ref):
    gdim = w1_ref.shape[0] - h_ref.shape[1] - 1
    w1h = w1_ref[gdim + 1:, :]
    wrho = w1_ref[gdim:gdim + 1, :]
    c_ref[...] = (
        jnp.dot(h_ref[...], w1h, preferred_element_type=jnp.float32)
        + rho_ref[...] * wrho
        + b1_ref[...]
    )


def _mlp_body(g_ref, s_ref, w1_ref, w2_ref, b2_ref, a_ref):
    gdim = g_ref.shape[1]
    w1g = w1_ref[:gdim, :]
    z = jnp.dot(g_ref[...], w1g, preferred_element_type=jnp.float32)
    h = jnp.maximum(z + s_ref[...], 0.0)
    a = jnp.dot(h, w2_ref[...], preferred_element_type=jnp.float32)
    a_ref[...] = a + b2_ref[...]


def _make_gather(n, feat):
    per_w = _CC * _R
    tail = n - 31 * per_w - 12 * _R
    mesh = plsc.VectorSubcoreMesh(core_axis_name="c", subcore_axis_name="s")

    @functools.partial(
        pl.kernel,
        mesh=mesh,
        out_type=jax.ShapeDtypeStruct((n, feat), jnp.float32),
        scratch_types=[
            pltpu.VMEM((per_w,), jnp.int32),
            pltpu.VMEM((2, _R, feat), jnp.float32),
            pltpu.SemaphoreType.DMA,
            pltpu.SemaphoreType.DMA,
        ],
    )
    def gather_k(c_hbm, idx_hbm, out_hbm, idx_v, rows_v, sem0, sem1):
        cid = lax.axis_index("c")
        sid = lax.axis_index("s")
        wid = sid * 2 + cid
        base = wid * per_w
        last = wid == _NW - 1
        sems = (sem0, sem1)

        @pl.when(jnp.logical_not(last))
        def _():
            pltpu.sync_copy(idx_hbm.at[pl.ds(base, per_w)], idx_v)

        @pl.when(last)
        def _():
            pltpu.sync_copy(idx_hbm.at[pl.ds(base, 12 * _R + tail)],
                            idx_v.at[pl.ds(0, 12 * _R + tail)])

        def start(c):
            buf = rows_v.at[c % 2]
            cp = {}
            if c < 12:
                cp["full"] = pltpu.async_copy(
                    c_hbm.at[idx_v.at[pl.ds(c * _R, _R)]], buf, sems[c % 2])
                return cp

            @pl.when(jnp.logical_not(last))
            def _():
                cp["full"] = pltpu.async_copy(
                    c_hbm.at[idx_v.at[pl.ds(c * _R, _R)]], buf, sems[c % 2])

            if c == 12:
                @pl.when(last)
                def _():
                    cp["tail"] = pltpu.async_copy(
                        c_hbm.at[idx_v.at[pl.ds(12 * _R, tail)]],
                        buf.at[pl.ds(0, tail)], sems[c % 2])
            return cp

        def drain(c, cp):
            buf = rows_v.at[c % 2]
            if c < 12:
                cp["full"].wait()
                pltpu.sync_copy(buf, out_hbm.at[pl.ds(base + c * _R, _R)])
                return

            @pl.when(jnp.logical_not(last))
            def _():
                cp["full"].wait()
                pltpu.sync_copy(buf, out_hbm.at[pl.ds(base + c * _R, _R)])

            if c == 12:
                @pl.when(last)
                def _():
                    cp["tail"].wait()
                    pltpu.sync_copy(buf.at[pl.ds(0, tail)],
                                    out_hbm.at[pl.ds(base + 12 * _R, tail)])

        cps = {0: start(0)}
        for c in range(_CC):
            if c + 1 < _CC:
                cps[c + 1] = start(c + 1)
            drain(c, cps[c])

    return gather_k


def kernel(p, mu_k, rho_k, h_k, g_i, W1, b1, W2, b2):
    n = p.shape[0]
    k = mu_k.shape[0]
    f = h_k.shape[1]
    gdim = g_i.shape[1]
    nb = n // _BN

    muT = mu_k.T
    idx = pl.pallas_call(
        _argmin_body,
        grid=(nb,),
        in_specs=[
            pl.BlockSpec((_BN, 3), lambda i: (i, 0)),
            pl.BlockSpec((3, k), lambda i: (0, 0)),
        ],
        out_specs=pl.BlockSpec((_BN, 1), lambda i: (i, 0)),
        out_shape=jax.ShapeDtypeStruct((n, 1), jnp.int32),
    )(p, muT)

    bk = 512
    ctab = pl.pallas_call(
        _ctable_body,
        grid=(k // bk,),
        in_specs=[
            pl.BlockSpec((bk, f), lambda i: (i, 0)),
            pl.BlockSpec((bk, 1), lambda i: (i, 0)),
            pl.BlockSpec(W1.shape, lambda i: (0, 0)),
            pl.BlockSpec((1, f), lambda i: (0, 0)),
        ],
        out_specs=pl.BlockSpec((bk, f), lambda i: (i, 0)),
        out_shape=jax.ShapeDtypeStruct((k, f), jnp.float32),
    )(h_k, rho_k.reshape(k, 1), W1, b1.reshape(1, f))

    s = _make_gather(n, f)(ctab, idx.reshape(n))

    a2 = pl.pallas_call(
        _mlp_body,
        grid=(nb,),
        in_specs=[
            pl.BlockSpec((_BN, gdim), lambda i: (i, 0)),
            pl.BlockSpec((_BN, f), lambda i: (i, 0)),
            pl.BlockSpec(W1.shape, lambda i: (0, 0)),
            pl.BlockSpec((f, 1), lambda i: (0, 0)),
            pl.BlockSpec((1, 1), lambda i: (0, 0)),
        ],
        out_specs=pl.BlockSpec((_BN, 1), lambda i: (i, 0)),
        out_shape=jax.ShapeDtypeStruct((n, 1), jnp.float32),
    )(g_i, s, W1, W2, b2.reshape(1, 1))

    return a2[:, 0]


# speedup vs baseline: 1.0140x; 1.0140x over previous
import functools

import jax
import jax.numpy as jnp
from jax import lax
from jax.experimental import pallas as pl
from jax.experimental.pallas import tpu as pltpu
from jax.experimental.pallas import tpu_sc as plsc

_NW = 32
_CC = 14
_R = 112
_BN = 1000


def _argmin_ctable_body(p_ref, muT_ref, h_ref, rho_ref, w1_ref, b1_ref,
                        idx_ref, c_ref):
    i = pl.program_id(0)
    nsteps = 8

    @pl.when(i < nsteps)
    def _():
        _ctable_body(h_ref, rho_ref, w1_ref, b1_ref, c_ref)

    _argmin_body(p_ref, muT_ref, idx_ref)


def _argmin_body(p_ref, muT_ref, idx_ref):
    p = p_ref[...]
    muT = muT_ref[...]
    k = muT.shape[1]
    m2 = jnp.sum(muT * muT, axis=0, keepdims=True)
    p2 = jnp.sum(p * p, axis=1, keepdims=True)
    dotn = jnp.dot(p, -2.0 * muT, preferred_element_type=jnp.float32)
    d2 = jnp.maximum((p2 + m2) + dotn, 0.0)
    mind = jnp.min(d2, axis=1, keepdims=True)
    ids = lax.broadcasted_iota(jnp.int32, d2.shape, 1)
    idx = jnp.min(jnp.where(d2 == mind, ids, k), axis=1, keepdims=True)
    idx_ref[...] = jnp.minimum(idx, k - 1).astype(jnp.int32)


def _ctable_body(h_ref, rho_ref, w1_ref, b1_ref, c_ref):
    gdim = w1_ref.shape[0] - h_ref.shape[1] - 1
    w1h = w1_ref[gdim + 1:, :]
    wrho = w1_ref[gdim:gdim + 1, :]
    c_ref[...] = (
        jnp.dot(h_ref[...], w1h, preferred_element_type=jnp.float32)
        + rho_ref[...] * wrho
        + b1_ref[...]
    )


def _mlp_body(g_ref, s_ref, w1_ref, w2_ref, b2_ref, a_ref):
    gdim = g_ref.shape[1]
    w1g = w1_ref[:gdim, :]
    z = jnp.dot(g_ref[...], w1g, preferred_element_type=jnp.float32)
    h = jnp.maximum(z + s_ref[...], 0.0)
    a = jnp.dot(h, w2_ref[...], preferred_element_type=jnp.float32)
    a_ref[...] = a + b2_ref[...]


def _make_gather(n, feat):
    per_w = _CC * _R
    tail = n - 31 * per_w - 12 * _R
    mesh = plsc.VectorSubcoreMesh(core_axis_name="c", subcore_axis_name="s")

    @functools.partial(
        pl.kernel,
        mesh=mesh,
        out_type=jax.ShapeDtypeStruct((n, feat), jnp.float32),
        scratch_types=[
            pltpu.VMEM((per_w,), jnp.int32),
            pltpu.VMEM((2, _R, feat), jnp.float32),
            pltpu.SemaphoreType.DMA,
            pltpu.SemaphoreType.DMA,
        ],
    )
    def gather_k(c_hbm, idx_hbm, out_hbm, idx_v, rows_v, sem0, sem1):
        cid = lax.axis_index("c")
        sid = lax.axis_index("s")
        wid = sid * 2 + cid
        base = wid * per_w
        last = wid == _NW - 1
        sems = (sem0, sem1)

        @pl.when(jnp.logical_not(last))
        def _():
            pltpu.sync_copy(idx_hbm.at[pl.ds(base, per_w)], idx_v)

        @pl.when(last)
        def _():
            pltpu.sync_copy(idx_hbm.at[pl.ds(base, 12 * _R + tail)],
                            idx_v.at[pl.ds(0, 12 * _R + tail)])

        def start(c):
            buf = rows_v.at[c % 2]
            cp = {}
            if c < 12:
                cp["full"] = pltpu.async_copy(
                    c_hbm.at[idx_v.at[pl.ds(c * _R, _R)]], buf, sems[c % 2])
                return cp

            @pl.when(jnp.logical_not(last))
            def _():
                cp["full"] = pltpu.async_copy(
                    c_hbm.at[idx_v.at[pl.ds(c * _R, _R)]], buf, sems[c % 2])

            if c == 12:
                @pl.when(last)
                def _():
                    cp["tail"] = pltpu.async_copy(
                        c_hbm.at[idx_v.at[pl.ds(12 * _R, tail)]],
                        buf.at[pl.ds(0, tail)], sems[c % 2])
            return cp

        def drain(c, cp):
            buf = rows_v.at[c % 2]
            if c < 12:
                cp["full"].wait()
                pltpu.sync_copy(buf, out_hbm.at[pl.ds(base + c * _R, _R)])
                return

            @pl.when(jnp.logical_not(last))
            def _():
                cp["full"].wait()
                pltpu.sync_copy(buf, out_hbm.at[pl.ds(base + c * _R, _R)])

            if c == 12:
                @pl.when(last)
                def _():
                    cp["tail"].wait()
                    pltpu.sync_copy(buf.at[pl.ds(0, tail)],
                                    out_hbm.at[pl.ds(base + 12 * _R, tail)])

        cps = {0: start(0)}
        for c in range(_CC):
            if c + 1 < _CC:
                cps[c + 1] = start(c + 1)
            drain(c, cps[c])

    return gather_k


def kernel(p, mu_k, rho_k, h_k, g_i, W1, b1, W2, b2):
    n = p.shape[0]
    k = mu_k.shape[0]
    f = h_k.shape[1]
    gdim = g_i.shape[1]
    nb = n // _BN

    muT = mu_k.T
    bk = 512
    nsteps = k // bk
    cmap = lambda i: (jnp.minimum(i, nsteps - 1), 0)
    idx, ctab = pl.pallas_call(
        _argmin_ctable_body,
        grid=(nb,),
        in_specs=[
            pl.BlockSpec((_BN, 3), lambda i: (i, 0)),
            pl.BlockSpec((3, k), lambda i: (0, 0)),
            pl.BlockSpec((bk, f), cmap),
            pl.BlockSpec((bk, 1), cmap),
            pl.BlockSpec(W1.shape, lambda i: (0, 0)),
            pl.BlockSpec((1, f), lambda i: (0, 0)),
        ],
        out_specs=[
            pl.BlockSpec((_BN, 1), lambda i: (i, 0)),
            pl.BlockSpec((bk, f), cmap),
        ],
        out_shape=[
            jax.ShapeDtypeStruct((n, 1), jnp.int32),
            jax.ShapeDtypeStruct((k, f), jnp.float32),
        ],
    )(p, muT, h_k, rho_k.reshape(k, 1), W1, b1.reshape(1, f))

    s = _make_gather(n, f)(ctab, idx.reshape(n))

    a2 = pl.pallas_call(
        _mlp_body,
        grid=(nb,),
        in_specs=[
            pl.BlockSpec((_BN, gdim), lambda i: (i, 0)),
            pl.BlockSpec((_BN, f), lambda i: (i, 0)),
            pl.BlockSpec(W1.shape, lambda i: (0, 0)),
            pl.BlockSpec((f, 1), lambda i: (0, 0)),
            pl.BlockSpec((1, 1), lambda i: (0, 0)),
        ],
        out_specs=pl.BlockSpec((_BN, 1), lambda i: (i, 0)),
        out_shape=jax.ShapeDtypeStruct((n, 1), jnp.float32),
    )(g_i, s, W1, W2, b2.reshape(1, 1))

    return a2[:, 0]
